# Initial kernel scaffold; baseline (speedup 1.0000x reference)
import jax
import jax.numpy as jnp
from jax.experimental import pallas as pl


def kernel(x, edge_index, edge_attr, batch, W1, b1, W2, b2, We, gamma, beta):
    raise NotImplementedError("write your pallas kernel here")


# speedup vs baseline: 3.1412x; 3.1412x over previous
import functools

import jax
import jax.numpy as jnp
from jax import lax
from jax.experimental import pallas as pl
from jax.experimental.pallas import tpu as pltpu
from jax.experimental.pallas import tpu_sc as plsc

_NC = 2
_NS = 16
_LANES = 16
_BLK = 128


def _edge_proj(edge_attr, We_l):
    E, DE = edge_attr.shape
    H = We_l.shape[1]
    BE = 2560
    assert E % BE == 0

    def body(a_ref, w_ref, o_ref):
        o_ref[...] = lax.dot_general(
            a_ref[...], w_ref[...], (((1,), (0,)), ((), ())),
            preferred_element_type=jnp.float32)

    return pl.pallas_call(
        body,
        grid=(E // BE,),
        in_specs=[pl.BlockSpec((BE, DE), lambda i: (i, 0)),
                  pl.BlockSpec((DE, H), lambda i: (0, 0))],
        out_specs=pl.BlockSpec((BE, H), lambda i: (i, 0)),
        out_shape=jax.ShapeDtypeStruct((E, H), jnp.float32),
    )(edge_attr, We_l)


@functools.cache
def _make_edge_agg(N, E, H):
    NW = _NC * _NS
    n_blocks = E // _BLK
    nb_per_w = (n_blocks + NW - 1) // NW
    n_full = N // _BLK
    rem = N - n_full * _BLK
    nch_per_tile = (n_full + _NS) // _NS
    mesh = plsc.VectorSubcoreMesh(core_axis_name="c", subcore_axis_name="s")

    @functools.partial(
        pl.kernel,
        mesh=mesh,
        out_type=jax.ShapeDtypeStruct((_NC, N, H), jnp.float32),
        scratch_types=[
            pltpu.VMEM((_BLK,), jnp.int32),
            pltpu.VMEM((_BLK,), jnp.int32),
            pltpu.VMEM((_BLK, H), jnp.float32),
            pltpu.VMEM((_BLK, H), jnp.float32),
            pltpu.VMEM_SHARED((N, H), jnp.float32),
            pltpu.SemaphoreType.DMA,
            pltpu.SemaphoreType.DMA,
        ],
    )
    def edge_agg(h_hbm, e_hbm, src_hbm, dst_hbm, out_hbm,
                 src_v, dst_v, hrows, erows, agg_sh, sem_g, sem_e):
        c = lax.axis_index("c")
        s = lax.axis_index("s")
        wid = c * _NS + s

        zvec = jnp.zeros((_LANES,), jnp.float32)

        @pl.loop(0, _BLK)
        def _(i):
            for j in range(H // _LANES):
                hrows[i, pl.ds(j * _LANES, _LANES)] = zvec

        @pl.loop(0, nch_per_tile)
        def _(k):
            ch = k * _NS + s

            @pl.when(ch < n_full)
            def _():
                pltpu.sync_copy(hrows, agg_sh.at[pl.ds(ch * _BLK, _BLK)])

            @pl.when(ch == n_full)
            def _():
                pltpu.sync_copy(hrows.at[pl.ds(0, rem)],
                                agg_sh.at[pl.ds(n_full * _BLK, rem)])

        plsc.subcore_barrier()

        @pl.loop(0, nb_per_w)
        def _(i):
            blk = i * NW + wid

            @pl.when(blk < n_blocks)
            def _():
                base = blk * _BLK
                pltpu.sync_copy(src_hbm.at[pl.ds(base, _BLK)], src_v)
                pltpu.sync_copy(dst_hbm.at[pl.ds(base, _BLK)], dst_v)
                cp_g = pltpu.async_copy(h_hbm.at[src_v], hrows, sem_g)
                cp_e = pltpu.async_copy(e_hbm.at[pl.ds(base, _BLK)], erows,
                                        sem_e)
                cp_g.wait()
                cp_e.wait()

                @pl.loop(0, _BLK)
                def _(k):
                    for j in range(H // _LANES):
                        sl = pl.ds(j * _LANES, _LANES)
                        hv = hrows[k, sl]
                        ev = erows[k, sl]
                        hrows[k, sl] = jnp.maximum(hv + ev, 0.0)

                pltpu.sync_copy(hrows, agg_sh.at[dst_v], add=True)

        plsc.subcore_barrier()

        @pl.loop(0, nch_per_tile)
        def _(k):
            ch = k * _NS + s

            @pl.when(ch < n_full)
            def _():
                pltpu.sync_copy(agg_sh.at[pl.ds(ch * _BLK, _BLK)],
                                out_hbm.at[c].at[pl.ds(ch * _BLK, _BLK)])

            @pl.when(ch == n_full)
            def _():
                pltpu.sync_copy(agg_sh.at[pl.ds(n_full * _BLK, rem)],
                                out_hbm.at[c].at[pl.ds(n_full * _BLK, rem)])

    return edge_agg


def _node_update(h_in, parts, W1l, b1l, W2l, b2l, gammal, betal, relu_out):
    N, H = h_in.shape

    def body(h_ref, p_ref, w1, b1, w2, b2, ga, be, o_ref):
        z = h_ref[...] + p_ref[0] + p_ref[1]
        u = lax.dot_general(z, w1[...], (((1,), (0,)), ((), ())),
                            preferred_element_type=jnp.float32) + b1[...]
        u = jnp.maximum(u, 0.0)
        v = lax.dot_general(u, w2[...], (((1,), (0,)), ((), ())),
                            preferred_element_type=jnp.float32) + b2[...]
        mu = jnp.mean(v, axis=0, keepdims=True)
        var = jnp.mean((v - mu) * (v - mu), axis=0, keepdims=True)
        zn = (v - mu) * lax.rsqrt(var + 1e-5) * ga[...] + be[...]
        if relu_out:
            zn = jnp.maximum(zn, 0.0)
        o_ref[...] = zn + h_ref[...]

    return pl.pallas_call(
        body,
        out_shape=jax.ShapeDtypeStruct((N, H), jnp.float32),
    )(h_in, parts, W1l, b1l, W2l, b2l, gammal, betal)


def _readout(h, batch_col, G):
    N, H = h.shape

    GB = 8

    def body(h_ref, b_ref, o_ref):
        g0 = pl.program_id(0) * GB
        hv = h_ref[...]
        bv = b_ref[...]
        rows = [jnp.max(jnp.where(bv == g0 + gg, hv, -jnp.inf),
                        axis=0, keepdims=True)
                for gg in range(GB)]
        o_ref[...] = jnp.concatenate(rows, axis=0)

    return pl.pallas_call(
        body,
        grid=(G // GB,),
        in_specs=[pl.BlockSpec((N, H), lambda g: (0, 0)),
                  pl.BlockSpec((N, 1), lambda g: (0, 0))],
        out_specs=pl.BlockSpec((GB, H), lambda g: (g, 0)),
        out_shape=jax.ShapeDtypeStruct((G, H), jnp.float32),
    )(h, batch_col)


def kernel(x, edge_index, edge_attr, batch, W1, b1, W2, b2, We, gamma, beta):
    N, H = x.shape
    E = edge_index.shape[1]
    L = W1.shape[0]
    G = 128

    src = edge_index[0]
    dst = edge_index[1]
    edge_agg = _make_edge_agg(N, E, H)

    es = [_edge_proj(edge_attr, We[l]) for l in range(L)]
    h = x
    for l in range(L):
        parts = edge_agg(h, es[l], src, dst)
        h = _node_update(h, parts,
                         W1[l], b1[l].reshape(1, -1),
                         W2[l], b2[l].reshape(1, -1),
                         gamma[l].reshape(1, -1), beta[l].reshape(1, -1),
                         relu_out=(l < L - 1))
    h_rep = _readout(h, batch.reshape(-1, 1), G)
    return h_rep, h


# speedup vs baseline: 4.2891x; 1.3654x over previous
import functools

import jax
import jax.numpy as jnp
from jax import lax
from jax.experimental import pallas as pl
from jax.experimental.pallas import tpu as pltpu
from jax.experimental.pallas import tpu_sc as plsc

_NC = 2
_NS = 16
_LANES = 16
_BLK = 128


def _edge_proj(edge_attr, We_l):
    E, DE = edge_attr.shape
    H = We_l.shape[1]
    BE = 2560
    assert E % BE == 0

    def body(a_ref, w_ref, o_ref):
        o_ref[...] = lax.dot_general(
            a_ref[...], w_ref[...], (((1,), (0,)), ((), ())),
            preferred_element_type=jnp.float32)

    return pl.pallas_call(
        body,
        grid=(E // BE,),
        in_specs=[pl.BlockSpec((BE, DE), lambda i: (i, 0)),
                  pl.BlockSpec((DE, H), lambda i: (0, 0))],
        out_specs=pl.BlockSpec((BE, H), lambda i: (i, 0)),
        out_shape=jax.ShapeDtypeStruct((E, H), jnp.float32),
    )(edge_attr, We_l)


@functools.cache
def _make_edge_agg(N, E, H):
    NW = _NC * _NS
    BLK = 80
    n_blocks = E // BLK
    bpt = n_blocks // NW
    assert n_blocks == bpt * NW
    zch = N // BLK
    assert zch * BLK == N
    zch_per_tile = (zch + _NS - 1) // _NS
    mesh = plsc.VectorSubcoreMesh(core_axis_name="c", subcore_axis_name="s")

    @functools.partial(
        pl.kernel,
        mesh=mesh,
        out_type=jax.ShapeDtypeStruct((_NC, N, H), jnp.float32),
        scratch_types=[
            pltpu.VMEM((2, BLK), jnp.int32),
            pltpu.VMEM((2, BLK), jnp.int32),
            pltpu.VMEM((2, BLK, H), jnp.float32),
            pltpu.VMEM((2, BLK, H), jnp.float32),
            pltpu.VMEM_SHARED((N, H), jnp.float32),
            pltpu.SemaphoreType.DMA,
            pltpu.SemaphoreType.DMA,
            pltpu.SemaphoreType.DMA,
            pltpu.SemaphoreType.DMA,
            pltpu.SemaphoreType.DMA,
            pltpu.SemaphoreType.DMA,
        ],
    )
    def edge_agg(h_hbm, e_hbm, src_hbm, dst_hbm, out_hbm,
                 src2, dst2, hrows2, erows2, agg_sh,
                 sem_i0, sem_i1, sem_g0, sem_g1, sem_e0, sem_e1):
        c = lax.axis_index("c")
        s = lax.axis_index("s")
        wid = c * _NS + s
        blk0 = wid * bpt
        sem_i = (sem_i0, sem_i1)
        sem_g = (sem_g0, sem_g1)
        sem_e = (sem_e0, sem_e1)

        zvec = jnp.zeros((_LANES,), jnp.float32)

        @pl.loop(0, BLK)
        def _(i):
            for j in range(H // _LANES):
                hrows2[0, i, pl.ds(j * _LANES, _LANES)] = zvec

        @pl.loop(0, zch_per_tile)
        def _(k):
            ch = k * _NS + s

            @pl.when(ch < zch)
            def _():
                pltpu.sync_copy(hrows2.at[0], agg_sh.at[pl.ds(ch * BLK, BLK)])

        plsc.subcore_barrier()

        def start_idx(slot, i):
            base = (blk0 + i) * BLK
            pltpu.async_copy(src_hbm.at[pl.ds(base, BLK)], src2.at[slot],
                             sem_i[slot])
            pltpu.async_copy(dst_hbm.at[pl.ds(base, BLK)], dst2.at[slot],
                             sem_i[slot])

        def wait_idx(slot):
            pltpu.make_async_copy(src_hbm.at[pl.ds(0, BLK)], src2.at[slot],
                                  sem_i[slot]).wait()
            pltpu.make_async_copy(dst_hbm.at[pl.ds(0, BLK)], dst2.at[slot],
                                  sem_i[slot]).wait()

        def start_data(slot, i):
            base = (blk0 + i) * BLK
            pltpu.async_copy(h_hbm.at[src2.at[slot]], hrows2.at[slot],
                             sem_g[slot])
            pltpu.async_copy(e_hbm.at[pl.ds(base, BLK)], erows2.at[slot],
                             sem_e[slot])

        def wait_data(slot):
            pltpu.make_async_copy(h_hbm.at[src2.at[slot]], hrows2.at[slot],
                                  sem_g[slot]).wait()
            pltpu.make_async_copy(e_hbm.at[pl.ds(0, BLK)], erows2.at[slot],
                                  sem_e[slot]).wait()

        def step(i, slot, nxt):
            wait_data(slot)

            @pl.when(i + 1 < bpt)
            def _():
                wait_idx(nxt)
                start_data(nxt, i + 1)

            @pl.loop(0, BLK)
            def _(k):
                for j in range(H // _LANES):
                    sl = pl.ds(j * _LANES, _LANES)
                    hv = hrows2[slot, k, sl]
                    ev = erows2[slot, k, sl]
                    hrows2[slot, k, sl] = jnp.maximum(hv + ev, 0.0)

            pltpu.sync_copy(hrows2.at[slot], agg_sh.at[dst2.at[slot]],
                            add=True)

            @pl.when(i + 2 < bpt)
            def _():
                start_idx(slot, i + 2)

        start_idx(0, 0)
        wait_idx(0)
        start_data(0, 0)
        start_idx(1, 1)

        @pl.loop(0, bpt // 2)
        def _(k):
            i = k * 2
            step(i, 0, 1)
            step(i + 1, 1, 0)

        if bpt % 2:
            step(bpt - 1, 0, 1)

        plsc.subcore_barrier()

        @pl.loop(0, zch_per_tile)
        def _(k):
            ch = k * _NS + s

            @pl.when(ch < zch)
            def _():
                pltpu.sync_copy(agg_sh.at[pl.ds(ch * BLK, BLK)],
                                out_hbm.at[c].at[pl.ds(ch * BLK, BLK)])

    return edge_agg


def _node_update(h_in, parts, W1l, b1l, W2l, b2l, gammal, betal, relu_out):
    N, H = h_in.shape

    def body(h_ref, p_ref, w1, b1, w2, b2, ga, be, o_ref):
        z = h_ref[...] + p_ref[0] + p_ref[1]
        u = lax.dot_general(z, w1[...], (((1,), (0,)), ((), ())),
                            preferred_element_type=jnp.float32) + b1[...]
        u = jnp.maximum(u, 0.0)
        v = lax.dot_general(u, w2[...], (((1,), (0,)), ((), ())),
                            preferred_element_type=jnp.float32) + b2[...]
        mu = jnp.mean(v, axis=0, keepdims=True)
        var = jnp.mean((v - mu) * (v - mu), axis=0, keepdims=True)
        zn = (v - mu) * lax.rsqrt(var + 1e-5) * ga[...] + be[...]
        if relu_out:
            zn = jnp.maximum(zn, 0.0)
        o_ref[...] = zn + h_ref[...]

    return pl.pallas_call(
        body,
        out_shape=jax.ShapeDtypeStruct((N, H), jnp.float32),
    )(h_in, parts, W1l, b1l, W2l, b2l, gammal, betal)


def _readout(h, batch_col, G):
    N, H = h.shape

    GB = 8

    def body(h_ref, b_ref, o_ref):
        g0 = pl.program_id(0) * GB
        hv = h_ref[...]
        bv = b_ref[...]
        rows = [jnp.max(jnp.where(bv == g0 + gg, hv, -jnp.inf),
                        axis=0, keepdims=True)
                for gg in range(GB)]
        o_ref[...] = jnp.concatenate(rows, axis=0)

    return pl.pallas_call(
        body,
        grid=(G // GB,),
        in_specs=[pl.BlockSpec((N, H), lambda g: (0, 0)),
                  pl.BlockSpec((N, 1), lambda g: (0, 0))],
        out_specs=pl.BlockSpec((GB, H), lambda g: (g, 0)),
        out_shape=jax.ShapeDtypeStruct((G, H), jnp.float32),
    )(h, batch_col)


def kernel(x, edge_index, edge_attr, batch, W1, b1, W2, b2, We, gamma, beta):
    N, H = x.shape
    E = edge_index.shape[1]
    L = W1.shape[0]
    G = 128

    src = edge_index[0]
    dst = edge_index[1]
    edge_agg = _make_edge_agg(N, E, H)

    es = [_edge_proj(edge_attr, We[l]) for l in range(L)]
    h = x
    for l in range(L):
        parts = edge_agg(h, es[l], src, dst)
        h = _node_update(h, parts,
                         W1[l], b1[l].reshape(1, -1),
                         W2[l], b2[l].reshape(1, -1),
                         gamma[l].reshape(1, -1), beta[l].reshape(1, -1),
                         relu_out=(l < L - 1))
    h_rep = _readout(h, batch.reshape(-1, 1), G)
    return h_rep, h
